# baseline (speedup 1.0000x reference)
import jax
import jax.numpy as jnp
from jax import lax
from jax.experimental import pallas as pl
from jax.experimental.pallas import tpu as pltpu
from jax.experimental.pallas import tpu_sc as plsc

_VOCAB = 1000000
_DIM = 64
_BATCH = 16384
_CTX = 10
_NEG = 20
_NW = 32
_BPW = _BATCH // _NW
_CB = 16
_NCH = _BPW // _CB
_SCORE_COLS = 32
_L = 16
_LINE = 2 * _DIM
_SPLIT = 512000
_PBLK = 4096
_NOFF = 48


def _repack_body(lo_ref, hi_ref, out_ref):
    eye = (lax.broadcasted_iota(jnp.int32, (_DIM, _DIM), 0)
           == lax.broadcasted_iota(jnp.int32, (_DIM, _DIM), 1)
           ).astype(jnp.float32)
    dn = (((0,), (0,)), ((), ()))
    out_ref[:, 0:_DIM] = lax.dot_general(
        lo_ref[...], eye, dn, preferred_element_type=jnp.float32)
    out_ref[:, _DIM:_LINE] = lax.dot_general(
        hi_ref[...], eye, dn, preferred_element_type=jnp.float32)


def _repack(table_t):
    nblk_v = (_VOCAB + _PBLK - 1) // _PBLK
    return pl.pallas_call(
        _repack_body,
        grid=(_SPLIT // _PBLK,),
        in_specs=[
            pl.BlockSpec((_DIM, _PBLK), lambda i: (0, i)),
            pl.BlockSpec((_DIM, _PBLK),
                         lambda i: (0, jnp.minimum(i + _SPLIT // _PBLK,
                                                   nblk_v - 1))),
        ],
        out_specs=pl.BlockSpec((_PBLK, _LINE), lambda i: (i, 0)),
        out_shape=jax.ShapeDtypeStruct((_SPLIT, _LINE), jnp.float32),
    )(table_t, table_t)


def _sc_scores_body(ctx_idx_hbm, cen_idx_hbm, neg_idx_hbm, offs_hbm,
                    tab_in_hbm, tab_out_hbm, scores_hbm,
                    idx_ctx, idx_cen, idx_neg,
                    rows_ctx, rows_cen, rows_neg, scores_v, offs_v, sem):
    nc = plsc.get_sparse_core_info().num_cores
    wid = lax.axis_index("s") * nc + lax.axis_index("c")
    tile_base = wid * _BPW

    def chunk_body(ch, carry):
        cbase = tile_base + ch * _CB
        pltpu.sync_copy(ctx_idx_hbm.at[pl.ds(cbase * _CTX, _CB * _CTX)],
                        idx_ctx)
        pltpu.sync_copy(cen_idx_hbm.at[pl.ds(cbase, _CB)], idx_cen)
        pltpu.sync_copy(neg_idx_hbm.at[pl.ds(cbase * _NEG, _CB * _NEG)],
                        idx_neg)
        pltpu.sync_copy(offs_hbm.at[pl.ds(cbase * _NOFF, _CB * _NOFF)],
                        offs_v)
        copies = []
        for off in range(0, _CB * _CTX, 128):
            n = min(128, _CB * _CTX - off)
            copies.append(pltpu.async_copy(
                tab_in_hbm.at[idx_ctx.at[pl.ds(off, n)]],
                rows_ctx.at[pl.ds(off, n)], sem))
        copies.append(pltpu.async_copy(tab_out_hbm.at[idx_cen], rows_cen,
                                       sem))
        for off in range(0, _CB * _NEG, 128):
            n = min(128, _CB * _NEG - off)
            copies.append(pltpu.async_copy(
                tab_out_hbm.at[idx_neg.at[pl.ds(off, n)]],
                rows_neg.at[pl.ds(off, n)], sem))
        for cp in copies:
            cp.wait()

        lane = lax.broadcasted_iota(jnp.int32, (_L,), 0)
        perms = [(lane + sh) % _L for sh in (8, 4, 2, 1)]
        lane_masks = [lane == i for i in range(_L)]
        dnums = lax.GatherDimensionNumbers(
            offset_dims=(), collapsed_slice_dims=(0,), start_index_map=(0,))

        def vperm(vec, p):
            return lax.gather(
                vec, p[:, None], dimension_numbers=dnums, slice_sizes=(1,),
                mode=lax.GatherScatterMode.PROMISE_IN_BOUNDS)

        def hsum16(vec):
            for p in perms:
                vec = vec + vperm(vec, p)
            return vec

        bcast_idx = [jnp.full((_L,), i, jnp.int32) for i in range(_L)]
        qcols = [lane + q * _L for q in range(_DIM // _L)]

        def half(rows_ref, r, off):
            return [rows_ref[r, pl.ds(off + q * _L, _L)]
                    for q in range(_DIM // _L)]

        def row_body(c, carry2):
            o1 = offs_v[pl.ds(c * _NOFF, _L)]
            o2 = offs_v[pl.ds(c * _NOFF + _L, _L)]
            s = [jnp.zeros((_L,), jnp.float32)] * (_DIM // _L)
            for j in range(_CTX):
                e = half(rows_ctx, c * _CTX + j, o1[j])
                for q in range(_DIM // _L):
                    s[q] = s[q] + e[q]
            s = [v * (1.0 / _CTX) for v in s]
            e = half(rows_cen, c, o1[_CTX])
            t = s[0] * e[0]
            for q in range(1, _DIM // _L):
                t = t + s[q] * e[q]
            out_lo = hsum16(t)
            out_hi = jnp.zeros((_L,), jnp.float32)
            for k in range(_NEG):
                ov = o2 if k < _L else offs_v[pl.ds(c * _NOFF + 2 * _L, _L)]
                e = half(rows_neg, c * _NEG + k, ov[k % _L])
                u = s[0] * e[0]
                for q in range(1, _DIM // _L):
                    u = u + s[q] * e[q]
                tot = hsum16(u)
                col = 1 + k
                if col < _L:
                    out_lo = jnp.where(lane_masks[col], tot, out_lo)
                else:
                    out_hi = jnp.where(lane_masks[col - _L], tot, out_hi)
            row_off = c * _SCORE_COLS
            scores_v[pl.ds(row_off, _L)] = out_lo
            scores_v[pl.ds(row_off + _L, _L)] = out_hi
            return carry2

        lax.fori_loop(0, _CB, row_body, 0)
        pltpu.sync_copy(
            scores_v,
            scores_hbm.at[pl.ds(cbase * _SCORE_COLS, _CB * _SCORE_COLS)])
        return carry

    lax.fori_loop(0, _NCH, chunk_body, 0)


def _loss_body(scores_ref, out_ref):
    s = scores_ref[...]
    col = lax.broadcasted_iota(jnp.int32, s.shape, 1)
    y = jnp.where(col == 0, s, -s)
    term = -jnp.log(jax.nn.sigmoid(y))
    term = jnp.where(col <= _NEG, term, 0.0)
    out_ref[...] = (jnp.sum(term) * (1.0 / _BATCH)).reshape(1, 1)


def kernel(context_words, center_word, negative_samples, in_embeddings,
           out_embeddings):
    ctx = context_words.astype(jnp.int32)
    cen = center_word.astype(jnp.int32)
    neg = negative_samples.astype(jnp.int32)
    ctx_line = (ctx >> 1).reshape(-1)
    cen_line = cen >> 1
    neg_line = (neg >> 1).reshape(-1)
    zeros5 = jnp.zeros((_BATCH, 5), jnp.int32)
    zeros12 = jnp.zeros((_BATCH, 12), jnp.int32)
    offs = jnp.concatenate(
        [(ctx & 1) * _DIM, ((cen & 1) * _DIM)[:, None], zeros5,
         (neg & 1) * _DIM, zeros12], axis=1).reshape(-1)

    tab_in = in_embeddings.reshape(_VOCAB // 2, _LINE)
    tab_out = out_embeddings.reshape(_VOCAB // 2, _LINE)

    mesh = plsc.VectorSubcoreMesh(core_axis_name="c", subcore_axis_name="s")
    scores = pl.kernel(
        _sc_scores_body,
        out_type=jax.ShapeDtypeStruct((_BATCH * _SCORE_COLS,), jnp.float32),
        mesh=mesh,
        scratch_types=[
            pltpu.VMEM((_CB * _CTX,), jnp.int32),
            pltpu.VMEM((_CB,), jnp.int32),
            pltpu.VMEM((_CB * _NEG,), jnp.int32),
            pltpu.VMEM((_CB * _CTX, _LINE), jnp.float32),
            pltpu.VMEM((_CB, _LINE), jnp.float32),
            pltpu.VMEM((_CB * _NEG, _LINE), jnp.float32),
            pltpu.VMEM((_CB * _SCORE_COLS,), jnp.float32),
            pltpu.VMEM((_CB * _NOFF,), jnp.int32),
            pltpu.SemaphoreType.DMA,
        ],
        compiler_params=pltpu.CompilerParams(use_tc_tiling_on_sc=False),
    )(ctx_line, cen_line, neg_line, offs, tab_in, tab_out)

    loss2d = pl.pallas_call(
        _loss_body,
        out_shape=jax.ShapeDtypeStruct((1, 1), jnp.float32),
    )(scores.reshape(_BATCH, _SCORE_COLS))
    return loss2d[0, 0]


# speedup vs baseline: 1.6230x; 1.6230x over previous
import jax
import jax.numpy as jnp
from jax import lax
from jax.experimental import pallas as pl
from jax.experimental.pallas import tpu as pltpu
from jax.experimental.pallas import tpu_sc as plsc

_VOCAB = 1000000
_DIM = 64
_BATCH = 16384
_CTX = 10
_NEG = 20
_NW = 32
_BPW = _BATCH // _NW
_CB = 16
_NCH = _BPW // _CB
_SCORE_COLS = 32
_L = 16
_LINE = 2 * _DIM
_SPLIT = 512000
_PBLK = 4096
_NOFF = 48


def _repack_body(lo_ref, hi_ref, out_ref):
    eye = (lax.broadcasted_iota(jnp.int32, (_DIM, _DIM), 0)
           == lax.broadcasted_iota(jnp.int32, (_DIM, _DIM), 1)
           ).astype(jnp.float32)
    dn = (((0,), (0,)), ((), ()))
    out_ref[:, 0:_DIM] = lax.dot_general(
        lo_ref[...], eye, dn, preferred_element_type=jnp.float32)
    out_ref[:, _DIM:_LINE] = lax.dot_general(
        hi_ref[...], eye, dn, preferred_element_type=jnp.float32)


def _repack(table_t):
    nblk_v = (_VOCAB + _PBLK - 1) // _PBLK
    return pl.pallas_call(
        _repack_body,
        grid=(_SPLIT // _PBLK,),
        in_specs=[
            pl.BlockSpec((_DIM, _PBLK), lambda i: (0, i)),
            pl.BlockSpec((_DIM, _PBLK),
                         lambda i: (0, jnp.minimum(i + _SPLIT // _PBLK,
                                                   nblk_v - 1))),
        ],
        out_specs=pl.BlockSpec((_PBLK, _LINE), lambda i: (i, 0)),
        out_shape=jax.ShapeDtypeStruct((_SPLIT, _LINE), jnp.float32),
    )(table_t, table_t)


def _sc_scores_body(ctx_idx_hbm, cen_idx_hbm, neg_idx_hbm, offs_hbm,
                    tab_in_hbm, tab_out_hbm, scores_hbm,
                    idx_ctx, idx_cen, idx_neg,
                    rows_ctx, rows_cen, rows_neg, scores_v, offs_v, sem):
    nc = plsc.get_sparse_core_info().num_cores
    wid = lax.axis_index("s") * nc + lax.axis_index("c")
    tile_base = wid * _BPW

    def chunk_body(ch, carry):
        cbase = tile_base + ch * _CB
        pltpu.sync_copy(ctx_idx_hbm.at[pl.ds(cbase * _CTX, _CB * _CTX)],
                        idx_ctx)
        pltpu.sync_copy(cen_idx_hbm.at[pl.ds(cbase, _CB)], idx_cen)
        pltpu.sync_copy(neg_idx_hbm.at[pl.ds(cbase * _NEG, _CB * _NEG)],
                        idx_neg)
        pltpu.sync_copy(offs_hbm.at[pl.ds(cbase * _NOFF, _CB * _NOFF)],
                        offs_v)
        copies = []
        for off in range(0, _CB * _CTX, 128):
            n = min(128, _CB * _CTX - off)
            copies.append(pltpu.async_copy(
                tab_in_hbm.at[idx_ctx.at[pl.ds(off, n)]],
                rows_ctx.at[pl.ds(off, n)], sem))
        copies.append(pltpu.async_copy(tab_out_hbm.at[idx_cen], rows_cen,
                                       sem))
        for off in range(0, _CB * _NEG, 128):
            n = min(128, _CB * _NEG - off)
            copies.append(pltpu.async_copy(
                tab_out_hbm.at[idx_neg.at[pl.ds(off, n)]],
                rows_neg.at[pl.ds(off, n)], sem))
        for cp in copies:
            cp.wait()

        lane = lax.broadcasted_iota(jnp.int32, (_L,), 0)
        perms = [(lane + sh) % _L for sh in (8, 4, 2, 1)]
        lane_masks = [lane == i for i in range(_L)]
        dnums = lax.GatherDimensionNumbers(
            offset_dims=(), collapsed_slice_dims=(0,), start_index_map=(0,))

        def vperm(vec, p):
            return lax.gather(
                vec, p[:, None], dimension_numbers=dnums, slice_sizes=(1,),
                mode=lax.GatherScatterMode.PROMISE_IN_BOUNDS)

        def hsum16(vec):
            for p in perms:
                vec = vec + vperm(vec, p)
            return vec

        bcast_idx = [jnp.full((_L,), i, jnp.int32) for i in range(_L)]
        qcols = [lane + q * _L for q in range(_DIM // _L)]

        def half(rows_ref, r, off):
            return [rows_ref[r, pl.ds(off + q * _L, _L)]
                    for q in range(_DIM // _L)]

        def row_body(c, carry2):
            o1 = offs_v[pl.ds(c * _NOFF, _L)]
            o2 = offs_v[pl.ds(c * _NOFF + _L, _L)]
            s = [jnp.zeros((_L,), jnp.float32)] * (_DIM // _L)
            for j in range(_CTX):
                e = half(rows_ctx, c * _CTX + j, o1[j])
                for q in range(_DIM // _L):
                    s[q] = s[q] + e[q]
            s = [v * (1.0 / _CTX) for v in s]
            e = half(rows_cen, c, o1[_CTX])
            t = s[0] * e[0]
            for q in range(1, _DIM // _L):
                t = t + s[q] * e[q]
            out_lo = hsum16(t)
            out_hi = jnp.zeros((_L,), jnp.float32)
            for k in range(_NEG):
                ov = o2 if k < _L else offs_v[pl.ds(c * _NOFF + 2 * _L, _L)]
                e = half(rows_neg, c * _NEG + k, ov[k % _L])
                u = s[0] * e[0]
                for q in range(1, _DIM // _L):
                    u = u + s[q] * e[q]
                tot = hsum16(u)
                col = 1 + k
                if col < _L:
                    out_lo = jnp.where(lane_masks[col], tot, out_lo)
                else:
                    out_hi = jnp.where(lane_masks[col - _L], tot, out_hi)
            row_off = c * _SCORE_COLS
            scores_v[pl.ds(row_off, _L)] = out_lo
            scores_v[pl.ds(row_off + _L, _L)] = out_hi
            return carry2

        lax.fori_loop(0, _CB, row_body, 0)
        pltpu.sync_copy(
            scores_v,
            scores_hbm.at[pl.ds(cbase * _SCORE_COLS, _CB * _SCORE_COLS)])
        return carry

    lax.fori_loop(0, _NCH, chunk_body, 0)


def _loss_body(scores_ref, out_ref):
    s = scores_ref[...]
    col = lax.broadcasted_iota(jnp.int32, s.shape, 1)
    y = jnp.where(col == 0, s, -s)
    term = -jnp.log(jax.nn.sigmoid(y))
    term = jnp.where(col <= _NEG, term, 0.0)
    out_ref[...] = (jnp.sum(term) * (1.0 / _BATCH)).reshape(1, 1)


def kernel(context_words, center_word, negative_samples, in_embeddings,
           out_embeddings):
    ctx = context_words.astype(jnp.int32)
    cen = center_word.astype(jnp.int32)
    neg = negative_samples.astype(jnp.int32)
    ctx_hi = (ctx >= _SPLIT).astype(jnp.int32)
    cen_hi = (cen >= _SPLIT).astype(jnp.int32)
    neg_hi = (neg >= _SPLIT).astype(jnp.int32)
    ctx_line = (ctx - ctx_hi * _SPLIT).reshape(-1)
    cen_line = cen - cen_hi * _SPLIT
    neg_line = (neg - neg_hi * _SPLIT).reshape(-1)
    zeros5 = jnp.zeros((_BATCH, 5), jnp.int32)
    zeros12 = jnp.zeros((_BATCH, 12), jnp.int32)
    offs = jnp.concatenate(
        [ctx_hi * _DIM, (cen_hi * _DIM)[:, None], zeros5,
         neg_hi * _DIM, zeros12], axis=1).reshape(-1)

    tab_in = _repack(in_embeddings.T)
    tab_out = _repack(out_embeddings.T)

    mesh = plsc.VectorSubcoreMesh(core_axis_name="c", subcore_axis_name="s")
    scores = pl.kernel(
        _sc_scores_body,
        out_type=jax.ShapeDtypeStruct((_BATCH * _SCORE_COLS,), jnp.float32),
        mesh=mesh,
        scratch_types=[
            pltpu.VMEM((_CB * _CTX,), jnp.int32),
            pltpu.VMEM((_CB,), jnp.int32),
            pltpu.VMEM((_CB * _NEG,), jnp.int32),
            pltpu.VMEM((_CB * _CTX, _LINE), jnp.float32),
            pltpu.VMEM((_CB, _LINE), jnp.float32),
            pltpu.VMEM((_CB * _NEG, _LINE), jnp.float32),
            pltpu.VMEM((_CB * _SCORE_COLS,), jnp.float32),
            pltpu.VMEM((_CB * _NOFF,), jnp.int32),
            pltpu.SemaphoreType.DMA,
        ],
        compiler_params=pltpu.CompilerParams(use_tc_tiling_on_sc=False),
    )(ctx_line, cen_line, neg_line, offs, tab_in, tab_out)

    loss2d = pl.pallas_call(
        _loss_body,
        out_shape=jax.ShapeDtypeStruct((1, 1), jnp.float32),
    )(scores.reshape(_BATCH, _SCORE_COLS))
    return loss2d[0, 0]


# speedup vs baseline: 1.6951x; 1.0444x over previous
import jax
import jax.numpy as jnp
from jax import lax
from jax.experimental import pallas as pl
from jax.experimental.pallas import tpu as pltpu
from jax.experimental.pallas import tpu_sc as plsc

_VOCAB = 1000000
_DIM = 64
_BATCH = 16384
_CTX = 10
_NEG = 20
_NW = 32
_BPW = _BATCH // _NW
_CB = 8
_NCH = _BPW // _CB
_SCORE_COLS = 32
_L = 16
_LINE = 2 * _DIM
_SPLIT = 512000
_PBLK = 4096
_NOFF = 48


def _repack_body(lo_ref, hi_ref, out_ref):
    eye = (lax.broadcasted_iota(jnp.int32, (_DIM, _DIM), 0)
           == lax.broadcasted_iota(jnp.int32, (_DIM, _DIM), 1)
           ).astype(jnp.float32)
    dn = (((0,), (0,)), ((), ()))
    out_ref[:, 0:_DIM] = lax.dot_general(
        lo_ref[...], eye, dn, preferred_element_type=jnp.float32)
    out_ref[:, _DIM:_LINE] = lax.dot_general(
        hi_ref[...], eye, dn, preferred_element_type=jnp.float32)


def _repack(table_t):
    nblk_v = (_VOCAB + _PBLK - 1) // _PBLK
    return pl.pallas_call(
        _repack_body,
        grid=(_SPLIT // _PBLK,),
        in_specs=[
            pl.BlockSpec((_DIM, _PBLK), lambda i: (0, i)),
            pl.BlockSpec((_DIM, _PBLK),
                         lambda i: (0, jnp.minimum(i + _SPLIT // _PBLK,
                                                   nblk_v - 1))),
        ],
        out_specs=pl.BlockSpec((_PBLK, _LINE), lambda i: (i, 0)),
        out_shape=jax.ShapeDtypeStruct((_SPLIT, _LINE), jnp.float32),
    )(table_t, table_t)


def _sc_scores_body(ctx_idx_hbm, cen_idx_hbm, neg_idx_hbm, offs_hbm,
                    tab_in_hbm, tab_out_hbm, scores_hbm,
                    idx_ctx0, idx_cen0, idx_neg0, offs_v0,
                    rows_ctx0, rows_cen0, rows_neg0, scores_v0,
                    idx_ctx1, idx_cen1, idx_neg1, offs_v1,
                    rows_ctx1, rows_cen1, rows_neg1, scores_v1,
                    sem_g0, sem_g1, sem_s0, sem_s1):
    nc = plsc.get_sparse_core_info().num_cores
    wid = lax.axis_index("s") * nc + lax.axis_index("c")
    tile_base = wid * _BPW
    bufs = [
        (idx_ctx0, idx_cen0, idx_neg0, offs_v0, rows_ctx0, rows_cen0,
         rows_neg0, scores_v0, sem_g0, sem_s0),
        (idx_ctx1, idx_cen1, idx_neg1, offs_v1, rows_ctx1, rows_cen1,
         rows_neg1, scores_v1, sem_g1, sem_s1),
    ]

    def gather_list(ch, b):
        idx_ctx, idx_cen, idx_neg, offs_v, rows_ctx, rows_cen, rows_neg, \
            scores_v, sem_g, sem_s = bufs[b]
        cbase = tile_base + ch * _CB
        pieces = [(tab_in_hbm, idx_ctx, rows_ctx, _CB * _CTX),
                  (tab_out_hbm, idx_cen, rows_cen, _CB),
                  (tab_out_hbm, idx_neg, rows_neg, _CB * _NEG)]
        out = []
        for tab, idx_v, rows_v, total in pieces:
            for off in range(0, total, 128):
                n = min(128, total - off)
                out.append((tab, idx_v.at[pl.ds(off, n)],
                            rows_v.at[pl.ds(off, n)], sem_g))
        return out

    def stage(ch, b):
        idx_ctx, idx_cen, idx_neg, offs_v, *_ = bufs[b]
        cbase = tile_base + ch * _CB
        pltpu.sync_copy(ctx_idx_hbm.at[pl.ds(cbase * _CTX, _CB * _CTX)],
                        idx_ctx)
        pltpu.sync_copy(cen_idx_hbm.at[pl.ds(cbase, _CB)], idx_cen)
        pltpu.sync_copy(neg_idx_hbm.at[pl.ds(cbase * _NEG, _CB * _NEG)],
                        idx_neg)
        pltpu.sync_copy(offs_hbm.at[pl.ds(cbase * _NOFF, _CB * _NOFF)],
                        offs_v)
        for tab, idx_s, rows_s, sem_g in gather_list(ch, b):
            pltpu.async_copy(tab.at[idx_s], rows_s, sem_g)

    def drain(ch, b):
        for tab, idx_s, rows_s, sem_g in gather_list(ch, b):
            pltpu.make_async_copy(tab.at[idx_s], rows_s, sem_g).wait()

    def compute(ch, b, h):
        idx_ctx, idx_cen, idx_neg, offs_v, rows_ctx, rows_cen, rows_neg, \
            scores_v, sem_g, sem_s = bufs[b]
        cbase = tile_base + ch * _CB
        out_slice = scores_hbm.at[
            pl.ds(cbase * _SCORE_COLS, _CB * _SCORE_COLS)]

        lane = lax.broadcasted_iota(jnp.int32, (_L,), 0)
        perms = [(lane + sh) % _L for sh in (8, 4, 2, 1)]
        lane_masks = [lane == i for i in range(_L)]
        dnums = lax.GatherDimensionNumbers(
            offset_dims=(), collapsed_slice_dims=(0,), start_index_map=(0,))

        def vperm(vec, p):
            return lax.gather(
                vec, p[:, None], dimension_numbers=dnums, slice_sizes=(1,),
                mode=lax.GatherScatterMode.PROMISE_IN_BOUNDS)

        def hsum16(vec):
            for p in perms:
                vec = vec + vperm(vec, p)
            return vec

        bcast_idx = [jnp.full((_L,), i, jnp.int32) for i in range(_L)]
        qcols = [lane + q * _L for q in range(_DIM // _L)]

        def half(rows_ref, r, off):
            return [rows_ref[r, pl.ds(off + q * _L, _L)]
                    for q in range(_DIM // _L)]

        def row_body(c, carry2):
            o1 = offs_v[pl.ds(c * _NOFF, _L)]
            o2 = offs_v[pl.ds(c * _NOFF + _L, _L)]
            s = [jnp.zeros((_L,), jnp.float32)] * (_DIM // _L)
            for j in range(_CTX):
                e = half(rows_ctx, c * _CTX + j, o1[j])
                for q in range(_DIM // _L):
                    s[q] = s[q] + e[q]
            s = [v * (1.0 / _CTX) for v in s]
            e = half(rows_cen, c, o1[_CTX])
            t = s[0] * e[0]
            for q in range(1, _DIM // _L):
                t = t + s[q] * e[q]
            out_lo = hsum16(t)
            out_hi = jnp.zeros((_L,), jnp.float32)
            for k in range(_NEG):
                ov = o2 if k < _L else offs_v[pl.ds(c * _NOFF + 2 * _L, _L)]
                e = half(rows_neg, c * _NEG + k, ov[k % _L])
                u = s[0] * e[0]
                for q in range(1, _DIM // _L):
                    u = u + s[q] * e[q]
                tot = hsum16(u)
                col = 1 + k
                if col < _L:
                    out_lo = jnp.where(lane_masks[col], tot, out_lo)
                else:
                    out_hi = jnp.where(lane_masks[col - _L], tot, out_hi)
            row_off = c * _SCORE_COLS
            scores_v[pl.ds(row_off, _L)] = out_lo
            scores_v[pl.ds(row_off + _L, _L)] = out_hi
            return carry2

        @pl.when(h > 0)
        def _():
            pltpu.make_async_copy(scores_v, out_slice, sem_s).wait()

        lax.fori_loop(0, _CB, row_body, 0)
        pltpu.async_copy(scores_v, out_slice, sem_s)

    stage(0, 0)

    def pipe_body(h, carry):
        ch0 = 2 * h
        stage(ch0 + 1, 1)
        drain(ch0, 0)
        compute(ch0, 0, h)

        @pl.when(h < _NCH // 2 - 1)
        def _():
            stage(ch0 + 2, 0)

        drain(ch0 + 1, 1)
        compute(ch0 + 1, 1, h)
        return carry

    lax.fori_loop(0, _NCH // 2, pipe_body, 0)
    tail = scores_hbm.at[pl.ds(0, _CB * _SCORE_COLS)]
    pltpu.make_async_copy(bufs[0][7], tail, bufs[0][9]).wait()
    pltpu.make_async_copy(bufs[1][7], tail, bufs[1][9]).wait()


def _loss_body(scores_ref, out_ref):
    s = scores_ref[...]
    col = lax.broadcasted_iota(jnp.int32, s.shape, 1)
    y = jnp.where(col == 0, s, -s)
    term = -jnp.log(jax.nn.sigmoid(y))
    term = jnp.where(col <= _NEG, term, 0.0)
    out_ref[...] = (jnp.sum(term) * (1.0 / _BATCH)).reshape(1, 1)


def kernel(context_words, center_word, negative_samples, in_embeddings,
           out_embeddings):
    ctx = context_words.astype(jnp.int32)
    cen = center_word.astype(jnp.int32)
    neg = negative_samples.astype(jnp.int32)
    ctx_hi = (ctx >= _SPLIT).astype(jnp.int32)
    cen_hi = (cen >= _SPLIT).astype(jnp.int32)
    neg_hi = (neg >= _SPLIT).astype(jnp.int32)
    ctx_line = (ctx - ctx_hi * _SPLIT).reshape(-1)
    cen_line = cen - cen_hi * _SPLIT
    neg_line = (neg - neg_hi * _SPLIT).reshape(-1)
    zeros5 = jnp.zeros((_BATCH, 5), jnp.int32)
    zeros12 = jnp.zeros((_BATCH, 12), jnp.int32)
    offs = jnp.concatenate(
        [ctx_hi * _DIM, (cen_hi * _DIM)[:, None], zeros5,
         neg_hi * _DIM, zeros12], axis=1).reshape(-1)

    tab_in = _repack(in_embeddings.T)
    tab_out = _repack(out_embeddings.T)

    mesh = plsc.VectorSubcoreMesh(core_axis_name="c", subcore_axis_name="s")
    scores = pl.kernel(
        _sc_scores_body,
        out_type=jax.ShapeDtypeStruct((_BATCH * _SCORE_COLS,), jnp.float32),
        mesh=mesh,
        scratch_types=(
            [pltpu.VMEM((_CB * _CTX,), jnp.int32),
             pltpu.VMEM((_CB,), jnp.int32),
             pltpu.VMEM((_CB * _NEG,), jnp.int32),
             pltpu.VMEM((_CB * _NOFF,), jnp.int32),
             pltpu.VMEM((_CB * _CTX, _LINE), jnp.float32),
             pltpu.VMEM((_CB, _LINE), jnp.float32),
             pltpu.VMEM((_CB * _NEG, _LINE), jnp.float32),
             pltpu.VMEM((_CB * _SCORE_COLS,), jnp.float32)] * 2
            + [pltpu.SemaphoreType.DMA] * 4),
        compiler_params=pltpu.CompilerParams(use_tc_tiling_on_sc=False),
    )(ctx_line, cen_line, neg_line, offs, tab_in, tab_out)

    loss2d = pl.pallas_call(
        _loss_body,
        out_shape=jax.ShapeDtypeStruct((1, 1), jnp.float32),
    )(scores.reshape(_BATCH, _SCORE_COLS))
    return loss2d[0, 0]


# speedup vs baseline: 1.8881x; 1.1139x over previous
import jax
import jax.numpy as jnp
from jax import lax
from jax.experimental import pallas as pl
from jax.experimental.pallas import tpu as pltpu
from jax.experimental.pallas import tpu_sc as plsc

_VOCAB = 1000000
_DIM = 64
_BATCH = 16384
_CTX = 10
_NEG = 20
_NW = 32
_BPW = _BATCH // _NW
_CB = 8
_NCH = _BPW // _CB
_SCORE_COLS = 32
_L = 16
_LINE = 2 * _DIM
_SPLIT = 512000
_PBLK = 10240
_NOFF = 48


def _repack_body(lo_ref, hi_ref, out_ref):
    eye = (lax.broadcasted_iota(jnp.int32, (_DIM, _DIM), 0)
           == lax.broadcasted_iota(jnp.int32, (_DIM, _DIM), 1)
           ).astype(jnp.float32)
    dn = (((0,), (0,)), ((), ()))
    out_ref[:, 0:_DIM] = lax.dot_general(
        lo_ref[...], eye, dn, preferred_element_type=jnp.float32)
    out_ref[:, _DIM:_LINE] = lax.dot_general(
        hi_ref[...], eye, dn, preferred_element_type=jnp.float32)


def _repack(table_t):
    nblk_v = (_VOCAB + _PBLK - 1) // _PBLK
    return pl.pallas_call(
        _repack_body,
        grid=(_SPLIT // _PBLK,),
        in_specs=[
            pl.BlockSpec((_DIM, _PBLK), lambda i: (0, i)),
            pl.BlockSpec((_DIM, _PBLK),
                         lambda i: (0, jnp.minimum(i + _SPLIT // _PBLK,
                                                   nblk_v - 1))),
        ],
        out_specs=pl.BlockSpec((_PBLK, _LINE), lambda i: (i, 0)),
        out_shape=jax.ShapeDtypeStruct((_SPLIT, _LINE), jnp.float32),
    )(table_t, table_t)


def _sc_scores_body(ctx_idx_hbm, cen_idx_hbm, neg_idx_hbm, offs_hbm,
                    tab_in_hbm, tab_out_hbm, scores_hbm,
                    idx_ctx0, idx_cen0, idx_neg0, offs_v0,
                    rows_ctx0, rows_cen0, rows_neg0, scores_v0,
                    idx_ctx1, idx_cen1, idx_neg1, offs_v1,
                    rows_ctx1, rows_cen1, rows_neg1, scores_v1,
                    sem_g0, sem_g1, sem_s0, sem_s1):
    nc = plsc.get_sparse_core_info().num_cores
    wid = lax.axis_index("s") * nc + lax.axis_index("c")
    tile_base = wid * _BPW
    bufs = [
        (idx_ctx0, idx_cen0, idx_neg0, offs_v0, rows_ctx0, rows_cen0,
         rows_neg0, scores_v0, sem_g0, sem_s0),
        (idx_ctx1, idx_cen1, idx_neg1, offs_v1, rows_ctx1, rows_cen1,
         rows_neg1, scores_v1, sem_g1, sem_s1),
    ]

    def gather_list(ch, b):
        idx_ctx, idx_cen, idx_neg, offs_v, rows_ctx, rows_cen, rows_neg, \
            scores_v, sem_g, sem_s = bufs[b]
        cbase = tile_base + ch * _CB
        pieces = [(tab_in_hbm, idx_ctx, rows_ctx, _CB * _CTX),
                  (tab_out_hbm, idx_cen, rows_cen, _CB),
                  (tab_out_hbm, idx_neg, rows_neg, _CB * _NEG)]
        out = []
        for tab, idx_v, rows_v, total in pieces:
            for off in range(0, total, 128):
                n = min(128, total - off)
                out.append((tab, idx_v.at[pl.ds(off, n)],
                            rows_v.at[pl.ds(off, n)], sem_g))
        return out

    def stage(ch, b):
        idx_ctx, idx_cen, idx_neg, offs_v, *_ = bufs[b]
        cbase = tile_base + ch * _CB
        pltpu.sync_copy(ctx_idx_hbm.at[pl.ds(cbase * _CTX, _CB * _CTX)],
                        idx_ctx)
        pltpu.sync_copy(cen_idx_hbm.at[pl.ds(cbase, _CB)], idx_cen)
        pltpu.sync_copy(neg_idx_hbm.at[pl.ds(cbase * _NEG, _CB * _NEG)],
                        idx_neg)
        pltpu.sync_copy(offs_hbm.at[pl.ds(cbase * _NOFF, _CB * _NOFF)],
                        offs_v)
        for tab, idx_s, rows_s, sem_g in gather_list(ch, b):
            pltpu.async_copy(tab.at[idx_s], rows_s, sem_g)

    def drain(ch, b):
        for tab, idx_s, rows_s, sem_g in gather_list(ch, b):
            pltpu.make_async_copy(tab.at[idx_s], rows_s, sem_g).wait()

    def compute(ch, b, h):
        idx_ctx, idx_cen, idx_neg, offs_v, rows_ctx, rows_cen, rows_neg, \
            scores_v, sem_g, sem_s = bufs[b]
        cbase = tile_base + ch * _CB
        out_slice = scores_hbm.at[
            pl.ds(cbase * _SCORE_COLS, _CB * _SCORE_COLS)]

        lane = lax.broadcasted_iota(jnp.int32, (_L,), 0)
        perms = [(lane + sh) % _L for sh in (8, 4, 2, 1)]
        lane_masks = [lane == i for i in range(_L)]
        dnums = lax.GatherDimensionNumbers(
            offset_dims=(), collapsed_slice_dims=(0,), start_index_map=(0,))

        def vperm(vec, p):
            return lax.gather(
                vec, p[:, None], dimension_numbers=dnums, slice_sizes=(1,),
                mode=lax.GatherScatterMode.PROMISE_IN_BOUNDS)

        def hsum16(vec):
            for p in perms:
                vec = vec + vperm(vec, p)
            return vec

        bcast_idx = [jnp.full((_L,), i, jnp.int32) for i in range(_L)]
        qcols = [lane + q * _L for q in range(_DIM // _L)]

        def half(rows_ref, r, off):
            return [rows_ref[r, pl.ds(off + q * _L, _L)]
                    for q in range(_DIM // _L)]

        def row_body(c, carry2):
            o1 = offs_v[pl.ds(c * _NOFF, _L)]
            o2 = offs_v[pl.ds(c * _NOFF + _L, _L)]
            s = [jnp.zeros((_L,), jnp.float32)] * (_DIM // _L)
            for j in range(_CTX):
                e = half(rows_ctx, c * _CTX + j, o1[j])
                for q in range(_DIM // _L):
                    s[q] = s[q] + e[q]
            s = [v * (1.0 / _CTX) for v in s]
            e = half(rows_cen, c, o1[_CTX])
            t = s[0] * e[0]
            for q in range(1, _DIM // _L):
                t = t + s[q] * e[q]
            out_lo = hsum16(t)
            out_hi = jnp.zeros((_L,), jnp.float32)
            for k in range(_NEG):
                ov = o2 if k < _L else offs_v[pl.ds(c * _NOFF + 2 * _L, _L)]
                e = half(rows_neg, c * _NEG + k, ov[k % _L])
                u = s[0] * e[0]
                for q in range(1, _DIM // _L):
                    u = u + s[q] * e[q]
                tot = hsum16(u)
                col = 1 + k
                if col < _L:
                    out_lo = jnp.where(lane_masks[col], tot, out_lo)
                else:
                    out_hi = jnp.where(lane_masks[col - _L], tot, out_hi)
            row_off = c * _SCORE_COLS
            scores_v[pl.ds(row_off, _L)] = out_lo
            scores_v[pl.ds(row_off + _L, _L)] = out_hi
            return carry2

        @pl.when(h > 0)
        def _():
            pltpu.make_async_copy(scores_v, out_slice, sem_s).wait()

        lax.fori_loop(0, _CB, row_body, 0)
        pltpu.async_copy(scores_v, out_slice, sem_s)

    stage(0, 0)

    def pipe_body(h, carry):
        ch0 = 2 * h
        stage(ch0 + 1, 1)
        drain(ch0, 0)
        compute(ch0, 0, h)

        @pl.when(h < _NCH // 2 - 1)
        def _():
            stage(ch0 + 2, 0)

        drain(ch0 + 1, 1)
        compute(ch0 + 1, 1, h)
        return carry

    lax.fori_loop(0, _NCH // 2, pipe_body, 0)
    tail = scores_hbm.at[pl.ds(0, _CB * _SCORE_COLS)]
    pltpu.make_async_copy(bufs[0][7], tail, bufs[0][9]).wait()
    pltpu.make_async_copy(bufs[1][7], tail, bufs[1][9]).wait()


def _loss_body(scores_ref, out_ref):
    s = scores_ref[...]
    col = lax.broadcasted_iota(jnp.int32, s.shape, 1)
    y = jnp.where(col == 0, s, -s)
    term = -jnp.log(jax.nn.sigmoid(y))
    term = jnp.where(col <= _NEG, term, 0.0)
    out_ref[...] = (jnp.sum(term) * (1.0 / _BATCH)).reshape(1, 1)


def kernel(context_words, center_word, negative_samples, in_embeddings,
           out_embeddings):
    ctx = context_words.astype(jnp.int32)
    cen = center_word.astype(jnp.int32)
    neg = negative_samples.astype(jnp.int32)
    ctx_hi = (ctx >= _SPLIT).astype(jnp.int32)
    cen_hi = (cen >= _SPLIT).astype(jnp.int32)
    neg_hi = (neg >= _SPLIT).astype(jnp.int32)
    ctx_line = (ctx - ctx_hi * _SPLIT).reshape(-1)
    cen_line = cen - cen_hi * _SPLIT
    neg_line = (neg - neg_hi * _SPLIT).reshape(-1)
    zeros5 = jnp.zeros((_BATCH, 5), jnp.int32)
    zeros12 = jnp.zeros((_BATCH, 12), jnp.int32)
    offs = jnp.concatenate(
        [ctx_hi * _DIM, (cen_hi * _DIM)[:, None], zeros5,
         neg_hi * _DIM, zeros12], axis=1).reshape(-1)

    tab_in = _repack(in_embeddings.T)
    tab_out = _repack(out_embeddings.T)

    mesh = plsc.VectorSubcoreMesh(core_axis_name="c", subcore_axis_name="s")
    scores = pl.kernel(
        _sc_scores_body,
        out_type=jax.ShapeDtypeStruct((_BATCH * _SCORE_COLS,), jnp.float32),
        mesh=mesh,
        scratch_types=(
            [pltpu.VMEM((_CB * _CTX,), jnp.int32),
             pltpu.VMEM((_CB,), jnp.int32),
             pltpu.VMEM((_CB * _NEG,), jnp.int32),
             pltpu.VMEM((_CB * _NOFF,), jnp.int32),
             pltpu.VMEM((_CB * _CTX, _LINE), jnp.float32),
             pltpu.VMEM((_CB, _LINE), jnp.float32),
             pltpu.VMEM((_CB * _NEG, _LINE), jnp.float32),
             pltpu.VMEM((_CB * _SCORE_COLS,), jnp.float32)] * 2
            + [pltpu.SemaphoreType.DMA] * 4),
        compiler_params=pltpu.CompilerParams(use_tc_tiling_on_sc=False),
    )(ctx_line, cen_line, neg_line, offs, tab_in, tab_out)

    loss2d = pl.pallas_call(
        _loss_body,
        out_shape=jax.ShapeDtypeStruct((1, 1), jnp.float32),
    )(scores.reshape(_BATCH, _SCORE_COLS))
    return loss2d[0, 0]


# speedup vs baseline: 1.9246x; 1.0193x over previous
import jax
import jax.numpy as jnp
from jax import lax
from jax.experimental import pallas as pl
from jax.experimental.pallas import tpu as pltpu
from jax.experimental.pallas import tpu_sc as plsc

_VOCAB = 1000000
_DIM = 64
_BATCH = 16384
_CTX = 10
_NEG = 20
_NW = 32
_BPW = _BATCH // _NW
_CB = 8
_NCH = _BPW // _CB
_SCORE_COLS = 32
_L = 16
_LINE = 2 * _DIM
_SPLIT = 512000
_PBLK = 20480
_NOFF = 48


def _repack_body(lo_ref, hi_ref, out_ref):
    eye = (lax.broadcasted_iota(jnp.int32, (_DIM, _DIM), 0)
           == lax.broadcasted_iota(jnp.int32, (_DIM, _DIM), 1)
           ).astype(jnp.float32)
    dn = (((0,), (0,)), ((), ()))
    out_ref[:, 0:_DIM] = lax.dot_general(
        lo_ref[...], eye, dn, preferred_element_type=jnp.float32)
    out_ref[:, _DIM:_LINE] = lax.dot_general(
        hi_ref[...], eye, dn, preferred_element_type=jnp.float32)


def _repack(table_t):
    nblk_v = (_VOCAB + _PBLK - 1) // _PBLK
    return pl.pallas_call(
        _repack_body,
        grid=(_SPLIT // _PBLK,),
        in_specs=[
            pl.BlockSpec((_DIM, _PBLK), lambda i: (0, i)),
            pl.BlockSpec((_DIM, _PBLK),
                         lambda i: (0, jnp.minimum(i + _SPLIT // _PBLK,
                                                   nblk_v - 1))),
        ],
        out_specs=pl.BlockSpec((_PBLK, _LINE), lambda i: (i, 0)),
        out_shape=jax.ShapeDtypeStruct((_SPLIT, _LINE), jnp.float32),
    )(table_t, table_t)


def _sc_scores_body(ctx_idx_hbm, cen_idx_hbm, neg_idx_hbm, offs_hbm,
                    tab_in_hbm, tab_out_hbm, scores_hbm,
                    idx_ctx0, idx_cen0, idx_neg0, offs_v0,
                    rows_ctx0, rows_cen0, rows_neg0, scores_v0,
                    idx_ctx1, idx_cen1, idx_neg1, offs_v1,
                    rows_ctx1, rows_cen1, rows_neg1, scores_v1,
                    sem_g0, sem_g1, sem_s0, sem_s1):
    nc = plsc.get_sparse_core_info().num_cores
    wid = lax.axis_index("s") * nc + lax.axis_index("c")
    tile_base = wid * _BPW
    bufs = [
        (idx_ctx0, idx_cen0, idx_neg0, offs_v0, rows_ctx0, rows_cen0,
         rows_neg0, scores_v0, sem_g0, sem_s0),
        (idx_ctx1, idx_cen1, idx_neg1, offs_v1, rows_ctx1, rows_cen1,
         rows_neg1, scores_v1, sem_g1, sem_s1),
    ]

    def gather_list(ch, b):
        idx_ctx, idx_cen, idx_neg, offs_v, rows_ctx, rows_cen, rows_neg, \
            scores_v, sem_g, sem_s = bufs[b]
        cbase = tile_base + ch * _CB
        pieces = [(tab_in_hbm, idx_ctx, rows_ctx, _CB * _CTX),
                  (tab_out_hbm, idx_cen, rows_cen, _CB),
                  (tab_out_hbm, idx_neg, rows_neg, _CB * _NEG)]
        out = []
        for tab, idx_v, rows_v, total in pieces:
            for off in range(0, total, 128):
                n = min(128, total - off)
                out.append((tab, idx_v.at[pl.ds(off, n)],
                            rows_v.at[pl.ds(off, n)], sem_g))
        return out

    def stage(ch, b):
        idx_ctx, idx_cen, idx_neg, offs_v, *_ = bufs[b]
        cbase = tile_base + ch * _CB
        pltpu.sync_copy(ctx_idx_hbm.at[pl.ds(cbase * _CTX, _CB * _CTX)],
                        idx_ctx)
        pltpu.sync_copy(cen_idx_hbm.at[pl.ds(cbase, _CB)], idx_cen)
        pltpu.sync_copy(neg_idx_hbm.at[pl.ds(cbase * _NEG, _CB * _NEG)],
                        idx_neg)
        pltpu.sync_copy(offs_hbm.at[pl.ds(cbase * _NOFF, _CB * _NOFF)],
                        offs_v)
        for tab, idx_s, rows_s, sem_g in gather_list(ch, b):
            pltpu.async_copy(tab.at[idx_s], rows_s, sem_g)

    def drain(ch, b):
        for tab, idx_s, rows_s, sem_g in gather_list(ch, b):
            pltpu.make_async_copy(tab.at[idx_s], rows_s, sem_g).wait()

    def compute(ch, b, h):
        idx_ctx, idx_cen, idx_neg, offs_v, rows_ctx, rows_cen, rows_neg, \
            scores_v, sem_g, sem_s = bufs[b]
        cbase = tile_base + ch * _CB
        out_slice = scores_hbm.at[
            pl.ds(cbase * _SCORE_COLS, _CB * _SCORE_COLS)]

        lane = lax.broadcasted_iota(jnp.int32, (_L,), 0)
        perms = [(lane + sh) % _L for sh in (8, 4, 2, 1)]
        lane_masks = [lane == i for i in range(_L)]
        dnums = lax.GatherDimensionNumbers(
            offset_dims=(), collapsed_slice_dims=(0,), start_index_map=(0,))

        def vperm(vec, p):
            return lax.gather(
                vec, p[:, None], dimension_numbers=dnums, slice_sizes=(1,),
                mode=lax.GatherScatterMode.PROMISE_IN_BOUNDS)

        def hsum16(vec):
            for p in perms:
                vec = vec + vperm(vec, p)
            return vec

        bcast_idx = [jnp.full((_L,), i, jnp.int32) for i in range(_L)]
        qcols = [lane + q * _L for q in range(_DIM // _L)]

        def half(rows_ref, r, off):
            return [rows_ref[r, pl.ds(off + q * _L, _L)]
                    for q in range(_DIM // _L)]

        def row_body(c, carry2):
            o1 = offs_v[pl.ds(c * _NOFF, _L)]
            o2 = offs_v[pl.ds(c * _NOFF + _L, _L)]
            s = [jnp.zeros((_L,), jnp.float32)] * (_DIM // _L)
            for j in range(_CTX):
                e = half(rows_ctx, c * _CTX + j, o1[j])
                for q in range(_DIM // _L):
                    s[q] = s[q] + e[q]
            s = [v * (1.0 / _CTX) for v in s]
            e = half(rows_cen, c, o1[_CTX])
            t = s[0] * e[0]
            for q in range(1, _DIM // _L):
                t = t + s[q] * e[q]
            out_lo = hsum16(t)
            out_hi = jnp.zeros((_L,), jnp.float32)
            for k in range(_NEG):
                ov = o2 if k < _L else offs_v[pl.ds(c * _NOFF + 2 * _L, _L)]
                e = half(rows_neg, c * _NEG + k, ov[k % _L])
                u = s[0] * e[0]
                for q in range(1, _DIM // _L):
                    u = u + s[q] * e[q]
                tot = hsum16(u)
                col = 1 + k
                if col < _L:
                    out_lo = jnp.where(lane_masks[col], tot, out_lo)
                else:
                    out_hi = jnp.where(lane_masks[col - _L], tot, out_hi)
            row_off = c * _SCORE_COLS
            scores_v[pl.ds(row_off, _L)] = out_lo
            scores_v[pl.ds(row_off + _L, _L)] = out_hi
            return carry2

        @pl.when(h > 0)
        def _():
            pltpu.make_async_copy(scores_v, out_slice, sem_s).wait()

        lax.fori_loop(0, _CB, row_body, 0)
        pltpu.async_copy(scores_v, out_slice, sem_s)

    stage(0, 0)

    def pipe_body(h, carry):
        ch0 = 2 * h
        stage(ch0 + 1, 1)
        drain(ch0, 0)
        compute(ch0, 0, h)

        @pl.when(h < _NCH // 2 - 1)
        def _():
            stage(ch0 + 2, 0)

        drain(ch0 + 1, 1)
        compute(ch0 + 1, 1, h)
        return carry

    lax.fori_loop(0, _NCH // 2, pipe_body, 0)
    tail = scores_hbm.at[pl.ds(0, _CB * _SCORE_COLS)]
    pltpu.make_async_copy(bufs[0][7], tail, bufs[0][9]).wait()
    pltpu.make_async_copy(bufs[1][7], tail, bufs[1][9]).wait()


def _loss_body(scores_ref, out_ref):
    s = scores_ref[...]
    col = lax.broadcasted_iota(jnp.int32, s.shape, 1)
    y = jnp.where(col == 0, s, -s)
    term = -jnp.log(jax.nn.sigmoid(y))
    term = jnp.where(col <= _NEG, term, 0.0)
    out_ref[...] = (jnp.sum(term) * (1.0 / _BATCH)).reshape(1, 1)


def kernel(context_words, center_word, negative_samples, in_embeddings,
           out_embeddings):
    ctx = context_words.astype(jnp.int32)
    cen = center_word.astype(jnp.int32)
    neg = negative_samples.astype(jnp.int32)
    ctx_hi = (ctx >= _SPLIT).astype(jnp.int32)
    cen_hi = (cen >= _SPLIT).astype(jnp.int32)
    neg_hi = (neg >= _SPLIT).astype(jnp.int32)
    ctx_line = (ctx - ctx_hi * _SPLIT).reshape(-1)
    cen_line = cen - cen_hi * _SPLIT
    neg_line = (neg - neg_hi * _SPLIT).reshape(-1)
    zeros5 = jnp.zeros((_BATCH, 5), jnp.int32)
    zeros12 = jnp.zeros((_BATCH, 12), jnp.int32)
    offs = jnp.concatenate(
        [ctx_hi * _DIM, (cen_hi * _DIM)[:, None], zeros5,
         neg_hi * _DIM, zeros12], axis=1).reshape(-1)

    tab_in = _repack(in_embeddings.T)
    tab_out = _repack(out_embeddings.T)

    mesh = plsc.VectorSubcoreMesh(core_axis_name="c", subcore_axis_name="s")
    scores = pl.kernel(
        _sc_scores_body,
        out_type=jax.ShapeDtypeStruct((_BATCH * _SCORE_COLS,), jnp.float32),
        mesh=mesh,
        scratch_types=(
            [pltpu.VMEM((_CB * _CTX,), jnp.int32),
             pltpu.VMEM((_CB,), jnp.int32),
             pltpu.VMEM((_CB * _NEG,), jnp.int32),
             pltpu.VMEM((_CB * _NOFF,), jnp.int32),
             pltpu.VMEM((_CB * _CTX, _LINE), jnp.float32),
             pltpu.VMEM((_CB, _LINE), jnp.float32),
             pltpu.VMEM((_CB * _NEG, _LINE), jnp.float32),
             pltpu.VMEM((_CB * _SCORE_COLS,), jnp.float32)] * 2
            + [pltpu.SemaphoreType.DMA] * 4),
        compiler_params=pltpu.CompilerParams(use_tc_tiling_on_sc=False),
    )(ctx_line, cen_line, neg_line, offs, tab_in, tab_out)

    loss2d = pl.pallas_call(
        _loss_body,
        out_shape=jax.ShapeDtypeStruct((1, 1), jnp.float32),
    )(scores.reshape(_BATCH, _SCORE_COLS))
    return loss2d[0, 0]


# speedup vs baseline: 2.0769x; 1.0792x over previous
import jax
import jax.numpy as jnp
from jax import lax
from jax.experimental import pallas as pl
from jax.experimental.pallas import tpu as pltpu
from jax.experimental.pallas import tpu_sc as plsc

_VOCAB = 1000000
_DIM = 64
_BATCH = 16384
_CTX = 10
_NEG = 20
_NW = 32
_BPW = _BATCH // _NW
_CB = 16
_NCH = _BPW // _CB
_SCORE_COLS = 32
_L = 16
_LINE = 2 * _DIM
_SPLIT = 512000
_PBLK = 20480
_NOFF = 48


def _repack_body(lo_ref, hi_ref, out_ref):
    eye = (lax.broadcasted_iota(jnp.int32, (_DIM, _DIM), 0)
           == lax.broadcasted_iota(jnp.int32, (_DIM, _DIM), 1)
           ).astype(jnp.float32)
    dn = (((0,), (0,)), ((), ()))
    out_ref[:, 0:_DIM] = lax.dot_general(
        lo_ref[...], eye, dn, preferred_element_type=jnp.float32)
    out_ref[:, _DIM:_LINE] = lax.dot_general(
        hi_ref[...], eye, dn, preferred_element_type=jnp.float32)


def _repack(table_t):
    nblk_v = (_VOCAB + _PBLK - 1) // _PBLK
    return pl.pallas_call(
        _repack_body,
        grid=(_SPLIT // _PBLK,),
        in_specs=[
            pl.BlockSpec((_DIM, _PBLK), lambda i: (0, i)),
            pl.BlockSpec((_DIM, _PBLK),
                         lambda i: (0, jnp.minimum(i + _SPLIT // _PBLK,
                                                   nblk_v - 1))),
        ],
        out_specs=pl.BlockSpec((_PBLK, _LINE), lambda i: (i, 0)),
        out_shape=jax.ShapeDtypeStruct((_SPLIT, _LINE), jnp.float32),
    )(table_t, table_t)


def _sc_scores_body(ctx_idx_hbm, cen_idx_hbm, neg_idx_hbm, offs_hbm,
                    tab_in_hbm, tab_out_hbm, scores_hbm,
                    idx_ctx0, idx_cen0, idx_neg0, offs_v0,
                    rows_ctx0, rows_cen0, rows_neg0, scores_v0,
                    idx_ctx1, idx_cen1, idx_neg1, offs_v1,
                    rows_ctx1, rows_cen1, rows_neg1, scores_v1,
                    sem_g0, sem_g1, sem_s0, sem_s1):
    nc = plsc.get_sparse_core_info().num_cores
    wid = lax.axis_index("s") * nc + lax.axis_index("c")
    tile_base = wid * _BPW
    bufs = [
        (idx_ctx0, idx_cen0, idx_neg0, offs_v0, rows_ctx0, rows_cen0,
         rows_neg0, scores_v0, sem_g0, sem_s0),
        (idx_ctx1, idx_cen1, idx_neg1, offs_v1, rows_ctx1, rows_cen1,
         rows_neg1, scores_v1, sem_g1, sem_s1),
    ]

    def gather_list(ch, b):
        idx_ctx, idx_cen, idx_neg, offs_v, rows_ctx, rows_cen, rows_neg, \
            scores_v, sem_g, sem_s = bufs[b]
        cbase = tile_base + ch * _CB
        pieces = [(tab_in_hbm, idx_ctx, rows_ctx, _CB * _CTX),
                  (tab_out_hbm, idx_cen, rows_cen, _CB),
                  (tab_out_hbm, idx_neg, rows_neg, _CB * _NEG)]
        out = []
        for tab, idx_v, rows_v, total in pieces:
            for off in range(0, total, 128):
                n = min(128, total - off)
                out.append((tab, idx_v.at[pl.ds(off, n)],
                            rows_v.at[pl.ds(off, n)], sem_g))
        return out

    def stage(ch, b):
        idx_ctx, idx_cen, idx_neg, offs_v, *_ = bufs[b]
        cbase = tile_base + ch * _CB
        pltpu.sync_copy(ctx_idx_hbm.at[pl.ds(cbase * _CTX, _CB * _CTX)],
                        idx_ctx)
        pltpu.sync_copy(cen_idx_hbm.at[pl.ds(cbase, _CB)], idx_cen)
        pltpu.sync_copy(neg_idx_hbm.at[pl.ds(cbase * _NEG, _CB * _NEG)],
                        idx_neg)
        pltpu.sync_copy(offs_hbm.at[pl.ds(cbase * _NOFF, _CB * _NOFF)],
                        offs_v)
        for tab, idx_s, rows_s, sem_g in gather_list(ch, b):
            pltpu.async_copy(tab.at[idx_s], rows_s, sem_g)

    def drain(ch, b):
        for tab, idx_s, rows_s, sem_g in gather_list(ch, b):
            pltpu.make_async_copy(tab.at[idx_s], rows_s, sem_g).wait()

    def compute(ch, b, h):
        idx_ctx, idx_cen, idx_neg, offs_v, rows_ctx, rows_cen, rows_neg, \
            scores_v, sem_g, sem_s = bufs[b]
        cbase = tile_base + ch * _CB
        out_slice = scores_hbm.at[
            pl.ds(cbase * _SCORE_COLS, _CB * _SCORE_COLS)]

        lane = lax.broadcasted_iota(jnp.int32, (_L,), 0)
        perms = [(lane + sh) % _L for sh in (8, 4, 2, 1)]
        lane_masks = [lane == i for i in range(_L)]
        dnums = lax.GatherDimensionNumbers(
            offset_dims=(), collapsed_slice_dims=(0,), start_index_map=(0,))

        def vperm(vec, p):
            return lax.gather(
                vec, p[:, None], dimension_numbers=dnums, slice_sizes=(1,),
                mode=lax.GatherScatterMode.PROMISE_IN_BOUNDS)

        def hsum16(vec):
            for p in perms:
                vec = vec + vperm(vec, p)
            return vec

        bcast_idx = [jnp.full((_L,), i, jnp.int32) for i in range(_L)]
        qcols = [lane + q * _L for q in range(_DIM // _L)]

        def half(rows_ref, r, off):
            return [rows_ref[r, pl.ds(off + q * _L, _L)]
                    for q in range(_DIM // _L)]

        def row_body(c, carry2):
            o1 = offs_v[pl.ds(c * _NOFF, _L)]
            o2 = offs_v[pl.ds(c * _NOFF + _L, _L)]
            s = [jnp.zeros((_L,), jnp.float32)] * (_DIM // _L)
            for j in range(_CTX):
                e = half(rows_ctx, c * _CTX + j, o1[j])
                for q in range(_DIM // _L):
                    s[q] = s[q] + e[q]
            s = [v * (1.0 / _CTX) for v in s]
            e = half(rows_cen, c, o1[_CTX])
            t = s[0] * e[0]
            for q in range(1, _DIM // _L):
                t = t + s[q] * e[q]
            out_lo = hsum16(t)
            out_hi = jnp.zeros((_L,), jnp.float32)
            for k in range(_NEG):
                ov = o2 if k < _L else offs_v[pl.ds(c * _NOFF + 2 * _L, _L)]
                e = half(rows_neg, c * _NEG + k, ov[k % _L])
                u = s[0] * e[0]
                for q in range(1, _DIM // _L):
                    u = u + s[q] * e[q]
                tot = hsum16(u)
                col = 1 + k
                if col < _L:
                    out_lo = jnp.where(lane_masks[col], tot, out_lo)
                else:
                    out_hi = jnp.where(lane_masks[col - _L], tot, out_hi)
            row_off = c * _SCORE_COLS
            scores_v[pl.ds(row_off, _L)] = out_lo
            scores_v[pl.ds(row_off + _L, _L)] = out_hi
            return carry2

        @pl.when(h > 0)
        def _():
            pltpu.make_async_copy(scores_v, out_slice, sem_s).wait()

        lax.fori_loop(0, _CB, row_body, 0)
        pltpu.async_copy(scores_v, out_slice, sem_s)

    stage(0, 0)

    def pipe_body(h, carry):
        ch0 = 2 * h
        stage(ch0 + 1, 1)
        drain(ch0, 0)
        compute(ch0, 0, h)

        @pl.when(h < _NCH // 2 - 1)
        def _():
            stage(ch0 + 2, 0)

        drain(ch0 + 1, 1)
        compute(ch0 + 1, 1, h)
        return carry

    lax.fori_loop(0, _NCH // 2, pipe_body, 0)
    tail = scores_hbm.at[pl.ds(0, _CB * _SCORE_COLS)]
    pltpu.make_async_copy(bufs[0][7], tail, bufs[0][9]).wait()
    pltpu.make_async_copy(bufs[1][7], tail, bufs[1][9]).wait()


def _loss_body(scores_ref, out_ref):
    s = scores_ref[...]
    col = lax.broadcasted_iota(jnp.int32, s.shape, 1)
    y = jnp.where(col == 0, s, -s)
    term = -jnp.log(jax.nn.sigmoid(y))
    term = jnp.where(col <= _NEG, term, 0.0)
    out_ref[...] = (jnp.sum(term) * (1.0 / _BATCH)).reshape(1, 1)


def kernel(context_words, center_word, negative_samples, in_embeddings,
           out_embeddings):
    ctx = context_words.astype(jnp.int32)
    cen = center_word.astype(jnp.int32)
    neg = negative_samples.astype(jnp.int32)
    ctx_hi = (ctx >= _SPLIT).astype(jnp.int32)
    cen_hi = (cen >= _SPLIT).astype(jnp.int32)
    neg_hi = (neg >= _SPLIT).astype(jnp.int32)
    ctx_line = (ctx - ctx_hi * _SPLIT).reshape(-1)
    cen_line = cen - cen_hi * _SPLIT
    neg_line = (neg - neg_hi * _SPLIT).reshape(-1)
    zeros5 = jnp.zeros((_BATCH, 5), jnp.int32)
    zeros12 = jnp.zeros((_BATCH, 12), jnp.int32)
    offs = jnp.concatenate(
        [ctx_hi * _DIM, (cen_hi * _DIM)[:, None], zeros5,
         neg_hi * _DIM, zeros12], axis=1).reshape(-1)

    tab_in = _repack(in_embeddings.T)
    tab_out = _repack(out_embeddings.T)

    mesh = plsc.VectorSubcoreMesh(core_axis_name="c", subcore_axis_name="s")
    scores = pl.kernel(
        _sc_scores_body,
        out_type=jax.ShapeDtypeStruct((_BATCH * _SCORE_COLS,), jnp.float32),
        mesh=mesh,
        scratch_types=(
            [pltpu.VMEM((_CB * _CTX,), jnp.int32),
             pltpu.VMEM((_CB,), jnp.int32),
             pltpu.VMEM((_CB * _NEG,), jnp.int32),
             pltpu.VMEM((_CB * _NOFF,), jnp.int32),
             pltpu.VMEM((_CB * _CTX, _LINE), jnp.float32),
             pltpu.VMEM((_CB, _LINE), jnp.float32),
             pltpu.VMEM((_CB * _NEG, _LINE), jnp.float32),
             pltpu.VMEM((_CB * _SCORE_COLS,), jnp.float32)] * 2
            + [pltpu.SemaphoreType.DMA] * 4),
        compiler_params=pltpu.CompilerParams(use_tc_tiling_on_sc=False),
    )(ctx_line, cen_line, neg_line, offs, tab_in, tab_out)

    loss2d = pl.pallas_call(
        _loss_body,
        out_shape=jax.ShapeDtypeStruct((1, 1), jnp.float32),
    )(scores.reshape(_BATCH, _SCORE_COLS))
    return loss2d[0, 0]
